# Initial kernel scaffold; baseline (speedup 1.0000x reference)
import jax
import jax.numpy as jnp
from jax.experimental import pallas as pl


def kernel(x, w1, g1, b1, w2, g2, b2, w3, g3, b3, w4, g4, b4, ws1, bs1, gs1, bes1, ws2, bs2, gs2, bes2, ws3, bs3):
    raise NotImplementedError("write your pallas kernel here")


# speedup vs baseline: 2.2327x; 2.2327x over previous
import jax
import jax.numpy as jnp
from jax.experimental import pallas as pl
from jax.experimental.pallas import tpu as pltpu

KNN = 20
EPS = 1e-5
R = 256
NEG = -3.0e38
BIGI = 2 ** 30


def _lrelu(v):
    return jnp.where(v > 0, v, 0.2 * v)


def _mm(a, b, dims):
    return jax.lax.dot_general(a, b, (dims, ((), ())),
                               preferred_element_type=jnp.float32)


def _stage_kernel(xcn_ref, xr_ref, xxr_ref, xxc_ref, wt_ref, g_ref, b_ref,
                  o_ref, pd_ref, f_ref):
    t = pl.program_id(1)
    n = xcn_ref.shape[2]
    c = xcn_ref.shape[1]
    xcn = xcn_ref[0]
    xtile = xcn_ref[0, :, pl.ds(t * R, R)]
    inner = -2.0 * _mm(xtile, xcn, ((0,), (0,)))
    pd_ref[...] = (-xxr_ref[0] - inner) - xxc_ref[0]
    xi = xr_ref[0, pl.ds(t * R, R), :]
    cols = jax.lax.broadcasted_iota(jnp.int32, (R, n), 1)

    def body(k, carry):
        dd = pd_ref[...]
        r = jnp.max(dd, axis=1, keepdims=True)
        cand = jnp.where(dd == r, cols, BIGI)
        jstar = jnp.min(cand, axis=1, keepdims=True)
        oh = cols == jstar
        pd_ref[...] = jnp.where(oh, NEG, dd)
        ohf = jnp.where(oh, 1.0, 0.0)
        xj = jax.lax.dot_general(ohf, xr_ref[0], (((1,), (0,)), ((), ())),
                                 preferred_element_type=jnp.float32,
                                 precision=jax.lax.Precision.HIGHEST)
        slab = jnp.concatenate([xj - xi, xi], axis=1)
        f_ref[:, pl.ds(k, 1), :] = slab[:, None, :]
        return carry

    jax.lax.fori_loop(0, KNN, body, 0)
    ff = f_ref[...].reshape(R * KNN, 2 * c)
    y = _mm(ff, wt_ref[...], ((1,), (0,)))
    y = y / jnp.sqrt(1.0 + EPS)
    y = y * g_ref[...] + b_ref[...]
    y = _lrelu(y)
    o_ref[0] = jnp.max(y.reshape(R, KNN, 64), axis=1)


def _run_stage(x_cn, w, gam, bet):
    b, c, n = x_cn.shape
    xx = jnp.sum(x_cn * x_cn, axis=1, keepdims=True)
    xxc = jnp.transpose(xx, (0, 2, 1))
    x_rows = jnp.transpose(x_cn, (0, 2, 1))
    wt = w.T
    return pl.pallas_call(
        _stage_kernel,
        grid=(b, n // R),
        in_specs=[
            pl.BlockSpec((1, c, n), lambda i, t: (i, 0, 0)),
            pl.BlockSpec((1, n, c), lambda i, t: (i, 0, 0)),
            pl.BlockSpec((1, 1, n), lambda i, t: (i, 0, 0)),
            pl.BlockSpec((1, R, 1), lambda i, t: (i, t, 0)),
            pl.BlockSpec((2 * c, 64), lambda i, t: (0, 0)),
            pl.BlockSpec((1, 64), lambda i, t: (0, 0)),
            pl.BlockSpec((1, 64), lambda i, t: (0, 0)),
        ],
        out_specs=pl.BlockSpec((1, R, 64), lambda i, t: (i, t, 0)),
        out_shape=jax.ShapeDtypeStruct((b, n, 64), jnp.float32),
        scratch_shapes=[
            pltpu.VMEM((R, n), jnp.float32),
            pltpu.VMEM((R, KNN, 2 * c), jnp.float32),
        ],
    )(x_cn, x_rows, xx, xxc, wt, gam[None, :], bet[None, :])


def _head_kernel(x1_ref, x2_ref, x3_ref, x4_ref, w1_ref, bb1_ref, g1_ref,
                 be1_ref, w2_ref, bb2_ref, g2_ref, be2_ref, w3_ref, bb3_ref,
                 o_ref):
    z = jnp.concatenate(
        [x1_ref[0], x2_ref[0], x3_ref[0], x4_ref[0]], axis=1)
    y = _mm(z, w1_ref[...], ((1,), (0,))) + bb1_ref[...]
    y = y / jnp.sqrt(1.0 + EPS)
    t1 = _lrelu(y * g1_ref[...] + be1_ref[...])
    y = _mm(t1, w2_ref[...], ((1,), (0,))) + bb2_ref[...]
    y = y / jnp.sqrt(1.0 + EPS)
    t2 = _lrelu(y * g2_ref[...] + be2_ref[...])
    o_ref[0] = _mm(t2, w3_ref[...], ((1,), (0,))) + bb3_ref[...]


def _run_head(x1, x2, x3, x4, ws1, bs1, gs1, bes1, ws2, bs2, gs2, bes2,
              ws3, bs3):
    b, n, _ = x1.shape
    ko = ws3.shape[0]
    fspec = lambda shape: pl.BlockSpec(shape, lambda i: (0, 0))
    xspec = pl.BlockSpec((1, n, 64), lambda i: (i, 0, 0))
    return pl.pallas_call(
        _head_kernel,
        grid=(b,),
        in_specs=[xspec, xspec, xspec, xspec,
                  fspec((256, 256)), fspec((1, 256)), fspec((1, 256)),
                  fspec((1, 256)),
                  fspec((256, 256)), fspec((1, 256)), fspec((1, 256)),
                  fspec((1, 256)),
                  fspec((256, ko)), fspec((1, ko))],
        out_specs=pl.BlockSpec((1, n, ko), lambda i: (i, 0, 0)),
        out_shape=jax.ShapeDtypeStruct((b, n, ko), jnp.float32),
    )(x1, x2, x3, x4,
      ws1.T, bs1[None, :], gs1[None, :], bes1[None, :],
      ws2.T, bs2[None, :], gs2[None, :], bes2[None, :],
      ws3.T, bs3[None, :])


def kernel(x, w1, g1, b1, w2, g2, b2, w3, g3, b3, w4, g4, b4, ws1, bs1,
           gs1, bes1, ws2, bs2, gs2, bes2, ws3, bs3):
    xt = jnp.transpose(x, (0, 2, 1))
    x1 = _run_stage(xt, w1, g1, b1)
    x2 = _run_stage(jnp.transpose(x1, (0, 2, 1)), w2, g2, b2)
    x3 = _run_stage(jnp.transpose(x2, (0, 2, 1)), w3, g3, b3)
    x4 = _run_stage(jnp.transpose(x3, (0, 2, 1)), w4, g4, b4)
    return _run_head(x1, x2, x3, x4, ws1, bs1, gs1, bes1,
                     ws2, bs2, gs2, bes2, ws3, bs3)


# speedup vs baseline: 3.5418x; 1.5864x over previous
import jax
import jax.numpy as jnp
from jax.experimental import pallas as pl
from jax.experimental.pallas import tpu as pltpu

KNN = 20
EPS = 1e-5
R = 256
NEG = -3.0e38
BIGI = 2 ** 30


def _lrelu(v):
    return jnp.where(v > 0, v, 0.2 * v)


def _mm(a, b, dims):
    return jax.lax.dot_general(a, b, (dims, ((), ())),
                               preferred_element_type=jnp.float32)


def _stage_kernel(xcn_ref, xr_ref, xxr_ref, xxc_ref, wt_ref, g_ref, b_ref,
                  o_ref, pd_ref, f_ref, xs_ref):
    t = pl.program_id(1)
    n = xcn_ref.shape[2]
    c = xcn_ref.shape[1]

    @pl.when(t == 0)
    def _():
        xr = xr_ref[0]
        bits = jax.lax.bitcast_convert_type(xr, jnp.int32)
        p1 = jax.lax.bitcast_convert_type(
            jnp.bitwise_and(bits, jnp.int32(-65536)), jnp.float32)
        r1 = xr - p1
        bits1 = jax.lax.bitcast_convert_type(r1, jnp.int32)
        p2 = jax.lax.bitcast_convert_type(
            jnp.bitwise_and(bits1, jnp.int32(-65536)), jnp.float32)
        xs_ref[0] = p1
        xs_ref[1] = p2
        xs_ref[2] = r1 - p2

    xcn = xcn_ref[0]
    xtile = xcn_ref[0, :, pl.ds(t * R, R)]
    inner = -2.0 * _mm(xtile, xcn, ((0,), (0,)))
    pd_ref[...] = (-xxr_ref[0] - inner) - xxc_ref[0]
    xi = xr_ref[0, pl.ds(t * R, R), :]
    cols = jax.lax.broadcasted_iota(jnp.int32, (R, n), 1)

    def body(k, carry):
        dd = pd_ref[...]
        r = jnp.max(dd, axis=1, keepdims=True)
        cand = jnp.where(dd == r, cols, BIGI)
        jstar = jnp.min(cand, axis=1, keepdims=True)
        oh = cols == jstar
        pd_ref[...] = jnp.where(oh, NEG, dd)
        ohf = jnp.where(oh, 1.0, 0.0)
        xj = (_mm(ohf, xs_ref[0], ((1,), (0,)))
              + _mm(ohf, xs_ref[1], ((1,), (0,)))
              + _mm(ohf, xs_ref[2], ((1,), (0,))))
        slab = jnp.concatenate([xj - xi, xi], axis=1)
        f_ref[:, pl.ds(k, 1), :] = slab[:, None, :]
        return carry

    jax.lax.fori_loop(0, KNN, body, 0)
    ff = f_ref[...].reshape(R * KNN, 2 * c)
    y = _mm(ff, wt_ref[...], ((1,), (0,)))
    y = y / jnp.sqrt(1.0 + EPS)
    y = y * g_ref[...] + b_ref[...]
    y = _lrelu(y)
    o_ref[0] = jnp.max(y.reshape(R, KNN, 64), axis=1)


def _run_stage(x_cn, w, gam, bet):
    b, c, n = x_cn.shape
    xx = jnp.sum(x_cn * x_cn, axis=1, keepdims=True)
    xxc = jnp.transpose(xx, (0, 2, 1))
    x_rows = jnp.transpose(x_cn, (0, 2, 1))
    wt = w.T
    return pl.pallas_call(
        _stage_kernel,
        grid=(b, n // R),
        in_specs=[
            pl.BlockSpec((1, c, n), lambda i, t: (i, 0, 0)),
            pl.BlockSpec((1, n, c), lambda i, t: (i, 0, 0)),
            pl.BlockSpec((1, 1, n), lambda i, t: (i, 0, 0)),
            pl.BlockSpec((1, R, 1), lambda i, t: (i, t, 0)),
            pl.BlockSpec((2 * c, 64), lambda i, t: (0, 0)),
            pl.BlockSpec((1, 64), lambda i, t: (0, 0)),
            pl.BlockSpec((1, 64), lambda i, t: (0, 0)),
        ],
        out_specs=pl.BlockSpec((1, R, 64), lambda i, t: (i, t, 0)),
        out_shape=jax.ShapeDtypeStruct((b, n, 64), jnp.float32),
        scratch_shapes=[
            pltpu.VMEM((R, n), jnp.float32),
            pltpu.VMEM((R, KNN, 2 * c), jnp.float32),
            pltpu.VMEM((3, n, c), jnp.float32),
        ],
    )(x_cn, x_rows, xx, xxc, wt, gam[None, :], bet[None, :])


def _head_kernel(x1_ref, x2_ref, x3_ref, x4_ref, w1_ref, bb1_ref, g1_ref,
                 be1_ref, w2_ref, bb2_ref, g2_ref, be2_ref, w3_ref, bb3_ref,
                 o_ref):
    z = jnp.concatenate(
        [x1_ref[0], x2_ref[0], x3_ref[0], x4_ref[0]], axis=1)
    y = _mm(z, w1_ref[...], ((1,), (0,))) + bb1_ref[...]
    y = y / jnp.sqrt(1.0 + EPS)
    t1 = _lrelu(y * g1_ref[...] + be1_ref[...])
    y = _mm(t1, w2_ref[...], ((1,), (0,))) + bb2_ref[...]
    y = y / jnp.sqrt(1.0 + EPS)
    t2 = _lrelu(y * g2_ref[...] + be2_ref[...])
    o_ref[0] = _mm(t2, w3_ref[...], ((1,), (0,))) + bb3_ref[...]


def _run_head(x1, x2, x3, x4, ws1, bs1, gs1, bes1, ws2, bs2, gs2, bes2,
              ws3, bs3):
    b, n, _ = x1.shape
    ko = ws3.shape[0]
    fspec = lambda shape: pl.BlockSpec(shape, lambda i: (0, 0))
    xspec = pl.BlockSpec((1, n, 64), lambda i: (i, 0, 0))
    return pl.pallas_call(
        _head_kernel,
        grid=(b,),
        in_specs=[xspec, xspec, xspec, xspec,
                  fspec((256, 256)), fspec((1, 256)), fspec((1, 256)),
                  fspec((1, 256)),
                  fspec((256, 256)), fspec((1, 256)), fspec((1, 256)),
                  fspec((1, 256)),
                  fspec((256, ko)), fspec((1, ko))],
        out_specs=pl.BlockSpec((1, n, ko), lambda i: (i, 0, 0)),
        out_shape=jax.ShapeDtypeStruct((b, n, ko), jnp.float32),
    )(x1, x2, x3, x4,
      ws1.T, bs1[None, :], gs1[None, :], bes1[None, :],
      ws2.T, bs2[None, :], gs2[None, :], bes2[None, :],
      ws3.T, bs3[None, :])


def kernel(x, w1, g1, b1, w2, g2, b2, w3, g3, b3, w4, g4, b4, ws1, bs1,
           gs1, bes1, ws2, bs2, gs2, bes2, ws3, bs3):
    xt = jnp.transpose(x, (0, 2, 1))
    x1 = _run_stage(xt, w1, g1, b1)
    x2 = _run_stage(jnp.transpose(x1, (0, 2, 1)), w2, g2, b2)
    x3 = _run_stage(jnp.transpose(x2, (0, 2, 1)), w3, g3, b3)
    x4 = _run_stage(jnp.transpose(x3, (0, 2, 1)), w4, g4, b4)
    return _run_head(x1, x2, x3, x4, ws1, bs1, gs1, bes1,
                     ws2, bs2, gs2, bes2, ws3, bs3)
